# Initial kernel scaffold; baseline (speedup 1.0000x reference)
import jax
import jax.numpy as jnp
from jax.experimental import pallas as pl


def kernel(x, edge_index, W1, b1, W3, b3):
    raise NotImplementedError("write your pallas kernel here")


# speedup vs baseline: 9.9395x; 9.9395x over previous
import functools

import jax
import jax.numpy as jnp
from jax import lax
from jax.experimental import pallas as pl
from jax.experimental.pallas import tpu as pltpu
from jax.experimental.pallas import tpu_sc as plsc

N_NODES = 10000
NPAD = 10240
STRIPE = NPAD // 16
N_EDGES = 320000
EPAD = 327680
TRASH = N_NODES

D_IN = 128
DIM = 128
ROWB = 1280
GRID_R = NPAD // ROWB

K1 = EPAD // 16 // 128
K2 = EPAD // 32 // 128
KD = EPAD // 32 // 128

_mesh = plsc.VectorSubcoreMesh(core_axis_name="c", subcore_axis_name="s")


@functools.partial(
    pl.kernel,
    out_type=jax.ShapeDtypeStruct((2 * NPAD,), jnp.float32),
    mesh=_mesh,
    scratch_types=[
        pltpu.VMEM((KD, 128), jnp.int32),
        pltpu.VMEM((128,), jnp.float32),
        pltpu.VMEM_SHARED((NPAD,), jnp.float32),
    ],
)
def _deg_kernel(zeros_hbm, dst_hbm, out_hbm, dst_v, ones_v, acc):
    c = lax.axis_index("c")
    t = lax.axis_index("s")
    pltpu.sync_copy(zeros_hbm, acc.at[pl.ds(t * STRIPE, STRIPE)])
    pltpu.sync_copy(dst_hbm.at[pl.ds((c * 16 + t) * KD, KD)], dst_v)
    for j in range(8):
        ones_v[pl.ds(j * 16, 16)] = jnp.ones((16,), jnp.float32)
    plsc.subcore_barrier()

    def body(i, carry):
        pltpu.sync_copy(ones_v, acc.at[dst_v.at[i]], add=True)
        return carry

    lax.fori_loop(0, KD, body, 0)
    plsc.subcore_barrier()
    pltpu.sync_copy(acc.at[pl.ds(t * STRIPE, STRIPE)],
                    out_hbm.at[pl.ds(c * NPAD + t * STRIPE, STRIPE)])


SUP = 16


def _make_scatter(k_chunks, src_stride_rows, dst_stride_rows):
    assert k_chunks % SUP == 0
    n_sup = k_chunks // SUP

    @functools.partial(
        pl.kernel,
        out_type=jax.ShapeDtypeStruct((2 * NPAD, 128), jnp.float32),
        mesh=_mesh,
        scratch_types=[
            pltpu.VMEM((2, SUP, 128), jnp.int32),
            pltpu.VMEM((2, SUP, 128), jnp.int32),
            pltpu.VMEM((2, 128, 128), jnp.float32),
            pltpu.VMEM_SHARED((NPAD, 128), jnp.float32),
            pltpu.SemaphoreType.DMA,
            pltpu.SemaphoreType.DMA,
            pltpu.SemaphoreType.DMA,
            pltpu.SemaphoreType.DMA,
        ],
    )
    def scat(zeros_hbm, src_hbm, dst_hbm, g_hbm, out_hbm,
             src_v, dst_v, rows, acc, semg0, semg1, semi0, semi1):
        c = lax.axis_index("c")
        t = lax.axis_index("s")
        src_base = c * src_stride_rows + t * k_chunks
        dst_base = c * dst_stride_rows + t * k_chunks
        semg = (semg0, semg1)
        semi = (semi0, semi1)

        def idx_copies(s):
            b = s % 2
            return (
                pltpu.make_async_copy(
                    src_hbm.at[pl.ds(src_base + s * SUP, SUP)], src_v.at[b], semi[b]),
                pltpu.make_async_copy(
                    dst_hbm.at[pl.ds(dst_base + s * SUP, SUP)], dst_v.at[b], semi[b]),
            )

        def idx_start(s):
            for cp in idx_copies(s):
                cp.start()

        def idx_wait(s):
            for cp in idx_copies(s):
                cp.wait()

        def g_copy(ci):
            b = ci % 2
            sref = src_v.at[(ci // SUP) % 2].at[ci % SUP]
            return pltpu.make_async_copy(g_hbm.at[sref], rows.at[b], semg[b])

        pltpu.sync_copy(zeros_hbm, acc.at[pl.ds(t * STRIPE, STRIPE)])
        idx_start(0)
        idx_wait(0)
        if n_sup > 1:
            idx_start(1)
        plsc.subcore_barrier()

        g_copy(0).start()
        for ci in range(k_chunks):
            nxt = ci + 1
            if nxt < k_chunks:
                if nxt % SUP == 0:
                    idx_wait(nxt // SUP)
                g_copy(nxt).start()
            g_copy(ci).wait()
            dref = dst_v.at[(ci // SUP) % 2].at[ci % SUP]
            pltpu.sync_copy(rows.at[ci % 2], acc.at[dref], add=True)
            if nxt % SUP == 0 and nxt // SUP + 1 < n_sup:
                idx_start(nxt // SUP + 1)

        plsc.subcore_barrier()
        pltpu.sync_copy(acc.at[pl.ds(t * STRIPE, STRIPE)],
                        out_hbm.at[pl.ds(c * NPAD + t * STRIPE, STRIPE)])

    return scat


_scatter1 = _make_scatter(K1, EPAD // 128, 0)
_scatter2 = _make_scatter(K2, EPAD // 256, EPAD // 256)


def _dis_from(deg_ref):
    deg = deg_ref[0:1, :] + deg_ref[1:2, :] + 1.0
    return lax.rsqrt(deg).reshape(ROWB)


def _tc1_body(deg_ref, x_ref, w_ref, out_ref):
    dis = _dis_from(deg_ref)
    h = jnp.dot(x_ref[...], w_ref[...], preferred_element_type=jnp.float32)
    out_ref[0] = h * dis[:, None]


def _tc2_body(deg_ref, tmp_ref, g1_ref, b1_ref, w3_ref, out_ref):
    dis = _dis_from(deg_ref)
    u0 = (tmp_ref[0] + g1_ref[0]) * dis[:, None] + b1_ref[0:1, :]
    u1 = (tmp_ref[1] + g1_ref[1]) * dis[:, None] + b1_ref[1:2, :]
    h0 = jnp.maximum(u0, 0.0)
    h1 = jnp.maximum(u1, 0.0)
    acc = (jnp.dot(h0, w3_ref[0], preferred_element_type=jnp.float32)
           + jnp.dot(h1, w3_ref[1], preferred_element_type=jnp.float32))
    out_ref[...] = acc * dis[:, None]


def _tc3_body(deg_ref, tmp_ref, g2_ref, b3_ref, out_ref):
    dis = _dis_from(deg_ref)
    out_ref[...] = (tmp_ref[0] + tmp_ref[1] + g2_ref[...]) * dis[:, None] + b3_ref[...]


_DEG_SPEC = pl.BlockSpec((2, ROWB), lambda i, *_: (0, i))
_DEG_SPEC1 = pl.BlockSpec((2, ROWB), lambda i: (0, i))


def _tc1(deg2, x_p, W1):
    return pl.pallas_call(
        _tc1_body,
        grid=(GRID_R, 2),
        in_specs=[
            _DEG_SPEC,
            pl.BlockSpec((ROWB, 128), lambda i, c: (i, 0)),
            pl.BlockSpec((128, 128), lambda i, c: (0, c)),
        ],
        out_specs=pl.BlockSpec((1, ROWB, 128), lambda i, c: (c, i, 0)),
        out_shape=jax.ShapeDtypeStruct((2, NPAD, 128), jnp.float32),
    )(deg2, x_p, W1)


def _tc2(deg2, tmp1, g1, b1_2, W3_3):
    return pl.pallas_call(
        _tc2_body,
        grid=(GRID_R,),
        in_specs=[
            _DEG_SPEC1,
            pl.BlockSpec((2, ROWB, 128), lambda i: (0, i, 0)),
            pl.BlockSpec((2, ROWB, 128), lambda i: (0, i, 0)),
            pl.BlockSpec((2, 128), lambda i: (0, 0)),
            pl.BlockSpec((2, 128, 128), lambda i: (0, 0, 0)),
        ],
        out_specs=pl.BlockSpec((ROWB, 128), lambda i: (i, 0)),
        out_shape=jax.ShapeDtypeStruct((NPAD, 128), jnp.float32),
    )(deg2, tmp1, g1, b1_2, W3_3)


def _tc3(deg2, tmp2, g2, b3_2):
    return pl.pallas_call(
        _tc3_body,
        grid=(GRID_R,),
        in_specs=[
            _DEG_SPEC1,
            pl.BlockSpec((2, ROWB, 128), lambda i: (0, i, 0)),
            pl.BlockSpec((ROWB, 128), lambda i: (i, 0)),
            pl.BlockSpec((1, 128), lambda i: (0, 0)),
        ],
        out_specs=pl.BlockSpec((ROWB, 128), lambda i: (i, 0)),
        out_shape=jax.ShapeDtypeStruct((NPAD, 128), jnp.float32),
    )(deg2, tmp2, g2, b3_2)


def kernel(x, edge_index, W1, b1, W3, b3):
    src = edge_index[0].astype(jnp.int32)
    dst = edge_index[1].astype(jnp.int32)
    pad = EPAD - N_EDGES
    src_p = jnp.concatenate([src, jnp.zeros((pad,), jnp.int32)])
    dst_p = jnp.concatenate([dst, jnp.full((pad,), TRASH, jnp.int32)])
    src_a = jnp.concatenate([src_p, src_p + NPAD]).reshape(2 * EPAD // 128, 128)
    src_2 = src_p.reshape(EPAD // 128, 128)
    dst_2 = dst_p.reshape(EPAD // 128, 128)
    x_p = jnp.pad(x, ((0, NPAD - N_NODES), (0, 0)))
    b1_2 = b1.reshape(2, 128)
    w3_3 = W3.reshape(2, 128, 128)
    b3_2 = b3.reshape(1, 128)
    zeros1 = jnp.zeros((STRIPE,), jnp.float32)
    zeros2 = jnp.zeros((STRIPE, 128), jnp.float32)

    deg_parts = _deg_kernel(zeros1, dst_2)
    deg2 = deg_parts.reshape(2, NPAD)

    g1 = _tc1(deg2, x_p, W1)
    tmp1 = _scatter1(zeros2, src_a, dst_2, g1.reshape(2 * NPAD, 128))
    g2 = _tc2(deg2, tmp1.reshape(2, NPAD, 128), g1, b1_2, w3_3)
    tmp2 = _scatter2(zeros2, src_2, dst_2, g2)
    out = _tc3(deg2, tmp2.reshape(2, NPAD, 128), g2, b3_2)
    return out[:N_NODES]


# speedup vs baseline: 11.7995x; 1.1871x over previous
import functools

import jax
import jax.numpy as jnp
from jax import lax
from jax.experimental import pallas as pl
from jax.experimental.pallas import tpu as pltpu
from jax.experimental.pallas import tpu_sc as plsc

N_NODES = 10000
NPAD = 10240
STRIPE = NPAD // 16
N_EDGES = 320000
EPAD = 327680
TRASH = N_NODES

D_IN = 128
DIM = 128
ROWB = 1280
GRID_R = NPAD // ROWB

K2 = EPAD // 32 // 128
KD = EPAD // 32 // 128

_mesh = plsc.VectorSubcoreMesh(core_axis_name="c", subcore_axis_name="s")


@functools.partial(
    pl.kernel,
    out_type=jax.ShapeDtypeStruct((2 * NPAD,), jnp.float32),
    mesh=_mesh,
    scratch_types=[
        pltpu.VMEM((KD, 128), jnp.int32),
        pltpu.VMEM((128,), jnp.float32),
        pltpu.VMEM_SHARED((NPAD,), jnp.float32),
    ],
)
def _deg_kernel(zeros_hbm, dst_hbm, out_hbm, dst_v, ones_v, acc):
    c = lax.axis_index("c")
    t = lax.axis_index("s")
    pltpu.sync_copy(zeros_hbm, acc.at[pl.ds(t * STRIPE, STRIPE)])
    pltpu.sync_copy(dst_hbm.at[pl.ds((c * 16 + t) * KD, KD)], dst_v)
    for j in range(8):
        ones_v[pl.ds(j * 16, 16)] = jnp.ones((16,), jnp.float32)
    plsc.subcore_barrier()

    def body(i, carry):
        pltpu.sync_copy(ones_v, acc.at[dst_v.at[i]], add=True)
        return carry

    lax.fori_loop(0, KD, body, 0)
    plsc.subcore_barrier()
    pltpu.sync_copy(acc.at[pl.ds(t * STRIPE, STRIPE)],
                    out_hbm.at[pl.ds(c * NPAD + t * STRIPE, STRIPE)])


SUP = 16


def _make_scatter(k_chunks, src_stride_rows, dst_stride_rows):
    assert k_chunks % SUP == 0
    n_sup = k_chunks // SUP

    @functools.partial(
        pl.kernel,
        out_type=jax.ShapeDtypeStruct((2 * NPAD, 128), jnp.float32),
        mesh=_mesh,
        scratch_types=[
            pltpu.VMEM((2, SUP, 128), jnp.int32),
            pltpu.VMEM((2, SUP, 128), jnp.int32),
            pltpu.VMEM((2, 128, 128), jnp.float32),
            pltpu.VMEM_SHARED((NPAD, 128), jnp.float32),
            pltpu.SemaphoreType.DMA,
            pltpu.SemaphoreType.DMA,
            pltpu.SemaphoreType.DMA,
            pltpu.SemaphoreType.DMA,
            pltpu.SemaphoreType.DMA,
            pltpu.SemaphoreType.DMA,
        ],
    )
    def scat(zeros_hbm, src_hbm, dst_hbm, g_hbm, out_hbm,
             src_v, dst_v, rows, acc, semg0, semg1, semi0, semi1, sems0, sems1):
        c = lax.axis_index("c")
        t = lax.axis_index("s")
        src_base = c * src_stride_rows + t * k_chunks
        dst_base = c * dst_stride_rows + t * k_chunks
        semg = (semg0, semg1)
        semi = (semi0, semi1)
        sems = (sems0, sems1)

        def idx_copies(s):
            b = s % 2
            return (
                pltpu.make_async_copy(
                    src_hbm.at[pl.ds(src_base + s * SUP, SUP)], src_v.at[b], semi[b]),
                pltpu.make_async_copy(
                    dst_hbm.at[pl.ds(dst_base + s * SUP, SUP)], dst_v.at[b], semi[b]),
            )

        def idx_start(s):
            for cp in idx_copies(s):
                cp.start()

        def idx_wait(s):
            for cp in idx_copies(s):
                cp.wait()

        def g_copy(ci):
            b = ci % 2
            sref = src_v.at[(ci // SUP) % 2].at[ci % SUP]
            return pltpu.make_async_copy(g_hbm.at[sref], rows.at[b], semg[b])

        pltpu.sync_copy(zeros_hbm, acc.at[pl.ds(t * STRIPE, STRIPE)])
        idx_start(0)
        idx_wait(0)
        if n_sup > 1:
            idx_start(1)
        plsc.subcore_barrier()

        def s_copy(ci):
            dref = dst_v.at[(ci // SUP) % 2].at[ci % SUP]
            return pltpu.make_async_copy(rows.at[ci % 2], acc.at[dref], sems[ci % 2])

        s_waited = set()

        def s_wait(i):
            if 0 <= i < k_chunks and i not in s_waited:
                s_waited.add(i)
                s_copy(i).wait()

        g_copy(0).start()
        for ci in range(k_chunks):
            nxt = ci + 1
            if nxt < k_chunks:
                if nxt % SUP == 0:
                    idx_wait(nxt // SUP)
                s_wait(nxt - 2)
                g_copy(nxt).start()
            g_copy(ci).wait()
            s_copy(ci).start(add=True)
            if nxt % SUP == 0 and nxt // SUP + 1 < n_sup:
                s_wait(ci - 1)
                s_wait(ci)
                idx_start(nxt // SUP + 1)
        s_wait(k_chunks - 2)
        s_wait(k_chunks - 1)

        plsc.subcore_barrier()
        pltpu.sync_copy(acc.at[pl.ds(t * STRIPE, STRIPE)],
                        out_hbm.at[pl.ds(c * NPAD + t * STRIPE, STRIPE)])

    return scat


_scatter = _make_scatter(K2, EPAD // 256, EPAD // 256)


def _dis_from(deg_ref):
    deg = deg_ref[0:1, :] + deg_ref[1:2, :] + 1.0
    return lax.rsqrt(deg).reshape(ROWB)


def _tc1_body(deg_ref, x_ref, out_ref):
    dis = _dis_from(deg_ref)
    out_ref[...] = x_ref[...] * dis[:, None]


def _tc2_body(deg_ref, tmp_ref, gx_ref, w1_ref, b1_ref, w3_ref, out_ref):
    dis = _dis_from(deg_ref)
    y = (tmp_ref[0] + tmp_ref[1] + gx_ref[...]) * dis[:, None]
    h = jnp.maximum(
        jnp.dot(y, w1_ref[...], preferred_element_type=jnp.float32) + b1_ref[...],
        0.0)
    g2 = jnp.dot(h, w3_ref[...], preferred_element_type=jnp.float32)
    out_ref[...] = g2 * dis[:, None]


def _tc3_body(deg_ref, tmp_ref, g2_ref, b3_ref, out_ref):
    dis = _dis_from(deg_ref)
    out_ref[...] = (tmp_ref[0] + tmp_ref[1] + g2_ref[...]) * dis[:, None] + b3_ref[...]


_DEG_SPEC = pl.BlockSpec((2, ROWB), lambda i, *_: (0, i))
_DEG_SPEC1 = pl.BlockSpec((2, ROWB), lambda i: (0, i))


def _tc1(deg2, x_p):
    return pl.pallas_call(
        _tc1_body,
        grid=(GRID_R,),
        in_specs=[
            _DEG_SPEC1,
            pl.BlockSpec((ROWB, 128), lambda i: (i, 0)),
        ],
        out_specs=pl.BlockSpec((ROWB, 128), lambda i: (i, 0)),
        out_shape=jax.ShapeDtypeStruct((NPAD, 128), jnp.float32),
    )(deg2, x_p)


def _tc2(deg2, tmp_x, g_x, W1, b1_2, W3):
    return pl.pallas_call(
        _tc2_body,
        grid=(GRID_R,),
        in_specs=[
            _DEG_SPEC1,
            pl.BlockSpec((2, ROWB, 128), lambda i: (0, i, 0)),
            pl.BlockSpec((ROWB, 128), lambda i: (i, 0)),
            pl.BlockSpec((128, 256), lambda i: (0, 0)),
            pl.BlockSpec((1, 256), lambda i: (0, 0)),
            pl.BlockSpec((256, 128), lambda i: (0, 0)),
        ],
        out_specs=pl.BlockSpec((ROWB, 128), lambda i: (i, 0)),
        out_shape=jax.ShapeDtypeStruct((NPAD, 128), jnp.float32),
    )(deg2, tmp_x, g_x, W1, b1_2, W3)


def _tc3(deg2, tmp2, g2, b3_2):
    return pl.pallas_call(
        _tc3_body,
        grid=(GRID_R,),
        in_specs=[
            _DEG_SPEC1,
            pl.BlockSpec((2, ROWB, 128), lambda i: (0, i, 0)),
            pl.BlockSpec((ROWB, 128), lambda i: (i, 0)),
            pl.BlockSpec((1, 128), lambda i: (0, 0)),
        ],
        out_specs=pl.BlockSpec((ROWB, 128), lambda i: (i, 0)),
        out_shape=jax.ShapeDtypeStruct((NPAD, 128), jnp.float32),
    )(deg2, tmp2, g2, b3_2)


def kernel(x, edge_index, W1, b1, W3, b3):
    src = edge_index[0].astype(jnp.int32)
    dst = edge_index[1].astype(jnp.int32)
    pad = EPAD - N_EDGES
    src_p = jnp.concatenate([src, jnp.zeros((pad,), jnp.int32)])
    dst_p = jnp.concatenate([dst, jnp.full((pad,), TRASH, jnp.int32)])
    src_2 = src_p.reshape(EPAD // 128, 128)
    dst_2 = dst_p.reshape(EPAD // 128, 128)
    x_p = jnp.pad(x, ((0, NPAD - N_NODES), (0, 0)))
    b1_2 = b1.reshape(1, 256)
    b3_2 = b3.reshape(1, 128)
    zeros1 = jnp.zeros((STRIPE,), jnp.float32)
    zeros2 = jnp.zeros((STRIPE, 128), jnp.float32)

    deg_parts = _deg_kernel(zeros1, dst_2)
    deg2 = deg_parts.reshape(2, NPAD)

    g_x = _tc1(deg2, x_p)
    tmp_x = _scatter(zeros2, src_2, dst_2, g_x)
    g2 = _tc2(deg2, tmp_x.reshape(2, NPAD, 128), g_x, W1, b1_2, W3)
    tmp2 = _scatter(zeros2, src_2, dst_2, g2)
    out = _tc3(deg2, tmp2.reshape(2, NPAD, 128), g2, b3_2)
    return out[:N_NODES]
